# baseline (speedup 1.0000x reference)
import functools

import jax
import jax.numpy as jnp
from jax import lax
from jax.experimental import pallas as pl
from jax.experimental.pallas import tpu as pltpu
from jax.experimental.pallas import tpu_sc as plsc

V = 32
D = 8
NC = 2
NS = 16
NW = NC * NS
R_REP = 64


def _m_table_body(e_ref, ht_ref, b_ref, m_ref):
    m_ref[...] = (
        jnp.dot(e_ref[...], ht_ref[...], preferred_element_type=jnp.float32)
        + b_ref[...]
    )


def _compute_m_table(embed_weight, head_weight_t, head_bias_row):
    return pl.pallas_call(
        _m_table_body,
        grid=(R_REP,),
        in_specs=[
            pl.BlockSpec((V, D), lambda r: (0, 0)),
            pl.BlockSpec((D, V), lambda r: (0, 0)),
            pl.BlockSpec((1, V), lambda r: (0, 0)),
        ],
        out_specs=pl.BlockSpec((V, V), lambda r: (r, 0)),
        out_shape=jax.ShapeDtypeStruct((R_REP * V, V), jnp.float32),
    )(embed_weight, head_weight_t, head_bias_row)


@functools.partial(jax.jit, static_argnames=("bt", "sl", "rows_per_chunk"))
def _sc_gather(m, ids, bt, sl, rows_per_chunk):
    rows_per_w = bt // NW
    n_chunks = rows_per_w // rows_per_chunk
    chunk = rows_per_chunk * sl
    mesh = plsc.VectorSubcoreMesh(core_axis_name="c", subcore_axis_name="s")

    @functools.partial(
        pl.kernel,
        mesh=mesh,
        out_type=jax.ShapeDtypeStruct((bt, sl, V), jnp.float32),
        scratch_types=[
            pltpu.VMEM((chunk,), jnp.int32),
            pltpu.VMEM((rows_per_chunk, sl, V), jnp.float32),
            pltpu.SemaphoreType.DMA,
        ],
        compiler_params=pltpu.CompilerParams(use_tc_tiling_on_sc=False),
    )
    def gather_kernel(m_hbm, idx_hbm, out_hbm, idx_v, rows_v, sem):
        wid = lax.axis_index("s") * NC + lax.axis_index("c")
        base_tok = wid * rows_per_w * sl
        base_row = wid * rows_per_w
        rep_off = lax.iota(jnp.int32, 16) * V
        for c in range(n_chunks):
            pltpu.sync_copy(idx_hbm.at[pl.ds(base_tok + c * chunk, chunk)], idx_v)

            def spread(i, carry):
                s = pl.ds(i * 16, 16)
                idx_v[s] = idx_v[s] + rep_off + (i % 4) * (16 * V)
                return carry

            lax.fori_loop(0, chunk // 16, spread, 0)
            copies = [
                pltpu.async_copy(
                    m_hbm.at[idx_v.at[pl.ds(j * sl, sl)]], rows_v.at[j], sem
                )
                for j in range(rows_per_chunk)
            ]
            for cp in copies:
                cp.wait()
            pltpu.sync_copy(
                rows_v, out_hbm.at[pl.ds(base_row + c * rows_per_chunk, rows_per_chunk)]
            )

    return gather_kernel(m, ids)


def kernel(input_ids, embed_weight, head_weight, head_bias):
    bt, sl = input_ids.shape
    b = bt * sl
    m = _compute_m_table(
        embed_weight, head_weight.T, head_bias.reshape(1, V)
    )
    ids = input_ids.reshape(b).astype(jnp.int32)
    return _sc_gather(m, ids, bt, sl, 16)


# speedup vs baseline: 1.4205x; 1.4205x over previous
import functools

import jax
import jax.numpy as jnp
from jax import lax
from jax.experimental import pallas as pl
from jax.experimental.pallas import tpu as pltpu
from jax.experimental.pallas import tpu_sc as plsc

V = 32
D = 8
NC = 2
NS = 16
NW = NC * NS


def _m_table_body(e8_ref, wb_ref, b_ref, m_ref):
    m_ref[...] = (
        jnp.dot(e8_ref[...], wb_ref[...], preferred_element_type=jnp.float32)
        + b_ref[...]
    )


def _compute_m8(embed_weight, head_weight, head_bias):
    e8 = embed_weight.reshape(8, 32)
    wb = jnp.kron(jnp.eye(4, dtype=jnp.float32), head_weight.T)
    b128 = jnp.tile(head_bias, 4).reshape(1, 128)
    return pl.pallas_call(
        _m_table_body,
        out_shape=jax.ShapeDtypeStruct((8, 128), jnp.float32),
    )(e8, wb, b128)


@functools.partial(jax.jit, static_argnames=("sl", "bt"))
def _sc_gather_t(m8, ids_t, sl, bt):
    lanes_per_w = bt // NW
    lch = 8
    n_chunks = sl // lch
    mesh = plsc.VectorSubcoreMesh(core_axis_name="c", subcore_axis_name="s")

    @functools.partial(
        pl.kernel,
        mesh=mesh,
        out_type=jax.ShapeDtypeStruct((sl, V, bt), jnp.float32),
        scratch_types=[
            pltpu.VMEM((8, 128), jnp.float32),
            pltpu.VMEM((lch, 128), jnp.int32),
            pltpu.VMEM((lch, V, 128), jnp.float32),
        ],
        compiler_params=pltpu.CompilerParams(
            use_tc_tiling_on_sc=True, needs_layout_passes=False
        ),
    )
    def gather_kernel(m_hbm, ids_hbm, out_hbm, m_v, ids_v, slab):
        wid = lax.axis_index("s") * NC + lax.axis_index("c")
        b0 = wid * lanes_per_w
        pltpu.sync_copy(m_hbm, m_v)

        def chunk_body(c, carry):
            pltpu.sync_copy(
                ids_hbm.at[pl.ds(c * lch, lch), pl.ds(b0, lanes_per_w)], ids_v
            )

            @plsc.parallel_loop(0, lch * 8, 1, unroll=2)
            def group(it):
                li = it // 8
                g = it % 8
                ids16 = ids_v[li, pl.ds(g * 16, 16)]
                row16 = lax.shift_right_logical(ids16, 2)
                col16 = lax.shift_left(lax.bitwise_and(ids16, 3), 5)
                for v in range(V):
                    val = plsc.load_gather(m_v, [row16, col16 + v])
                    slab[li, v, pl.ds(g * 16, 16)] = val
            pltpu.sync_copy(
                slab,
                out_hbm.at[pl.ds(c * lch, lch), :, pl.ds(b0, lanes_per_w)],
            )
            return carry

        lax.fori_loop(0, n_chunks, chunk_body, 0)

    return gather_kernel(m8, ids_t)


def kernel(input_ids, embed_weight, head_weight, head_bias):
    bt, sl = input_ids.shape
    m8 = _compute_m8(embed_weight, head_weight, head_bias)
    ids_t = input_ids.T.astype(jnp.int32)
    p = _sc_gather_t(m8, ids_t, sl, bt)
    return p.transpose(2, 0, 1)
